# baseline (speedup 1.0000x reference)
import dataclasses
import functools

import jax
import jax.numpy as jnp
from jax import lax
from jax.experimental import pallas as pl
from jax.experimental.pallas import tpu as pltpu
from jax.experimental.pallas import tpu_sc as plsc

N = 16384
NNZ = 268435
D = 256

NC = 2
NS = 16
NW = NC * NS
L = 16

ROWS_PER_GROUP = 256
NUM_GROUPS = N // ROWS_PER_GROUP
GROUPS_PER_TILE = NUM_GROUPS // NW
WBUF = 4096
W = 64
NBUF = 2
NNZ_PAD = ((NNZ + WBUF + 7) // 8) * 8

_GATHER_DNUMS = lax.GatherDimensionNumbers(
    offset_dims=(), collapsed_slice_dims=(0,), start_index_map=(0,))


def _bcast_lane(v, idx):
    return lax.gather(v, idx[:, None], _GATHER_DNUMS, (1,),
                      mode=lax.GatherScatterMode.PROMISE_IN_BOUNDS)


def _sc_spmm(rows_p, cols_p, vals_p, zeros, weight):
    mesh = plsc.VectorSubcoreMesh(core_axis_name="c", subcore_axis_name="s")
    cp = pltpu.CompilerParams()
    if "needs_layout_passes" in pltpu.CompilerParams.__dataclass_fields__:
        cp = dataclasses.replace(cp, needs_layout_passes=False)

    @functools.partial(
        pl.kernel,
        compiler_params=cp,
        out_type=jax.ShapeDtypeStruct((N, D), jnp.float32),
        mesh=mesh,
        scratch_types=[
            pltpu.VMEM((WBUF,), jnp.int32),
            pltpu.VMEM((WBUF,), jnp.int32),
            pltpu.VMEM((WBUF,), jnp.float32),
            pltpu.VMEM((L,), jnp.int32),
            pltpu.VMEM((L,), jnp.int32),
            [pltpu.VMEM((W, D), jnp.float32) for _ in range(NBUF)],
            pltpu.VMEM((ROWS_PER_GROUP, D), jnp.float32),
            pltpu.SemaphoreType.DMA,
            [pltpu.SemaphoreType.DMA for _ in range(NBUF)],
        ],
    )
    def sc_kernel(rows_hbm, cols_hbm, vals_hbm, zeros_hbm, w_hbm, out_hbm,
                  rows_buf, cols_buf, vals_buf, probe_v, pval_v, g, acc_v,
                  sem0, sems):
        wid = lax.axis_index("s") * NC + lax.axis_index("c")
        lane = lax.broadcasted_iota(jnp.int32, (L,), 0)
        lane1 = lane + 1
        idx1 = [dj * L + lane for dj in range(D // L)]

        def search(boundary):
            lo = jnp.int32(-1)
            hi = jnp.int32(NNZ)
            for _ in range(6):
                span = hi - lo
                probes = lo + ((span * lane1 + 15) >> 4)
                probe_v[...] = jnp.minimum(probes, NNZ - 1)
                pltpu.async_copy(rows_hbm.at[probe_v], pval_v, sem0).wait()
                val = jnp.where(probes >= NNZ, N, pval_v[...])
                m = jnp.sum(jnp.where(val < boundary, 1, 0))
                new_lo = lo + ((span * m + 15) >> 4)
                hi = lo + ((span * (m + 1) + 15) >> 4)
                lo = new_lo
            return hi

        @pl.loop(0, GROUPS_PER_TILE)
        def _(cc):
            c = wid * GROUPS_PER_TILE + cc
            base_row = pl.multiple_of(c * ROWS_PER_GROUP, 8)
            start = search(base_row)
            end = search(base_row + ROWS_PER_GROUP)
            pltpu.sync_copy(zeros_hbm, acc_v)

            a0 = start - (start & 7)
            nsw = (end - a0 + WBUF - 1) // WBUF

            @pl.loop(0, nsw)
            def _(w):
                k0g = pl.multiple_of(a0 + w * WBUF, 8)
                h1 = pltpu.async_copy(
                    rows_hbm.at[pl.ds(k0g, WBUF)], rows_buf, sem0)
                h2 = pltpu.async_copy(
                    cols_hbm.at[pl.ds(k0g, WBUF)], cols_buf, sems[0])
                h3 = pltpu.async_copy(
                    vals_hbm.at[pl.ds(k0g, WBUF)], vals_buf, sems[1])
                h1.wait()
                h2.wait()
                h3.wait()
                wend = jnp.minimum(end - k0g, WBUF)
                nblk = (wend + W - 1) // W

                def issue(t, b):
                    tb = jnp.minimum(t, nblk - 1)
                    src = w_hbm.at[cols_buf.at[pl.ds(tb * W, W)]]
                    pltpu.async_copy(src, g[b], sems[b])

                def wait(b):
                    pltpu.make_async_copy(
                        w_hbm.at[cols_buf.at[pl.ds(0, W)]], g[b],
                        sems[b]).wait()

                def compute(t, b):
                    tb = jnp.minimum(t, nblk - 1)
                    live = t < nblk
                    for g16 in range(W // L):
                        kbase = tb * W + g16 * L
                        rv = rows_buf[pl.ds(kbase, L)]
                        vv = vals_buf[pl.ds(kbase, L)]
                        pos = (k0g + kbase) + lane
                        valid = (pos >= start) & (pos < end) & live
                        v_eff = jnp.where(valid, vv, 0.0)
                        lr = jnp.clip(rv - base_row, 0, ROWS_PER_GROUP - 1)

                        @plsc.parallel_loop(0, L, unroll=4)
                        def _(j):
                            jf = jnp.zeros((L,), jnp.int32) + j
                            v_j = _bcast_lane(v_eff, jf)
                            lr_j = _bcast_lane(lr, jf)
                            gf = jf + g16 * L
                            for dj in range(D // L):
                                g16v = plsc.load_gather(g[b], [gf, idx1[dj]])
                                plsc.addupdate_scatter(
                                    acc_v, [lr_j, idx1[dj]], v_j * g16v)

                for b in range(NBUF):
                    issue(b, b)

                @pl.loop(0, (nblk + NBUF - 1) // NBUF)
                def _(u):
                    for b in range(NBUF):
                        t = u * NBUF + b
                        wait(b)
                        compute(t, b)
                        issue(t + NBUF, b)

                for b in range(NBUF):
                    wait(b)

            pltpu.sync_copy(acc_v, out_hbm.at[pl.ds(base_row, ROWS_PER_GROUP)])

    return sc_kernel(rows_p, cols_p, vals_p, zeros, weight)


def kernel(inp_rows, inp_cols, inp_values, weight):
    pad = NNZ_PAD - NNZ
    rows_p = jnp.pad(inp_rows, (0, pad), constant_values=N - 1)
    cols_p = jnp.pad(inp_cols, (0, pad), constant_values=0)
    vals_p = jnp.pad(inp_values, (0, pad), constant_values=0.0)
    zeros = jnp.zeros((ROWS_PER_GROUP, D), jnp.float32)
    return _sc_spmm(rows_p, cols_p, vals_p, zeros, weight)


# speedup vs baseline: 1.0111x; 1.0111x over previous
import dataclasses
import functools

import jax
import jax.numpy as jnp
from jax import lax
from jax.experimental import pallas as pl
from jax.experimental.pallas import tpu as pltpu
from jax.experimental.pallas import tpu_sc as plsc

N = 16384
NNZ = 268435
D = 256

NC = 2
NS = 16
NW = NC * NS
L = 16

ROWS_PER_GROUP = 256
NUM_GROUPS = N // ROWS_PER_GROUP
GROUPS_PER_TILE = NUM_GROUPS // NW
WBUF = 4096
W = 32
NBUF = 4
NNZ_PAD = ((NNZ + WBUF + 7) // 8) * 8

_GATHER_DNUMS = lax.GatherDimensionNumbers(
    offset_dims=(), collapsed_slice_dims=(0,), start_index_map=(0,))


def _bcast_lane(v, idx):
    return lax.gather(v, idx[:, None], _GATHER_DNUMS, (1,),
                      mode=lax.GatherScatterMode.PROMISE_IN_BOUNDS)


def _sc_spmm(rows_p, cols_p, vals_p, zeros, weight):
    mesh = plsc.VectorSubcoreMesh(core_axis_name="c", subcore_axis_name="s")
    cp = pltpu.CompilerParams()
    if "needs_layout_passes" in pltpu.CompilerParams.__dataclass_fields__:
        cp = dataclasses.replace(cp, needs_layout_passes=False)

    @functools.partial(
        pl.kernel,
        compiler_params=cp,
        out_type=jax.ShapeDtypeStruct((N, D), jnp.float32),
        mesh=mesh,
        scratch_types=[
            pltpu.VMEM((WBUF,), jnp.int32),
            pltpu.VMEM((WBUF,), jnp.int32),
            pltpu.VMEM((WBUF,), jnp.float32),
            pltpu.VMEM((L,), jnp.int32),
            pltpu.VMEM((L,), jnp.int32),
            [pltpu.VMEM((W, D), jnp.float32) for _ in range(NBUF)],
            pltpu.VMEM((ROWS_PER_GROUP, D), jnp.float32),
            pltpu.SemaphoreType.DMA,
            [pltpu.SemaphoreType.DMA for _ in range(NBUF)],
        ],
    )
    def sc_kernel(rows_hbm, cols_hbm, vals_hbm, zeros_hbm, w_hbm, out_hbm,
                  rows_buf, cols_buf, vals_buf, probe_v, pval_v, g, acc_v,
                  sem0, sems):
        wid = lax.axis_index("s") * NC + lax.axis_index("c")
        lane = lax.broadcasted_iota(jnp.int32, (L,), 0)
        lane1 = lane + 1
        idx1 = [dj * L + lane for dj in range(D // L)]

        tile_base = wid * GROUPS_PER_TILE * ROWS_PER_GROUP
        grp = jnp.minimum(lane // 5, 2)
        lane_local = lane - grp * 5 + 1
        b_lane = tile_base + grp * ROWS_PER_GROUP
        lo = [jnp.int32(-1)] * 3
        hi = [jnp.int32(NNZ)] * 3
        for _ in range(8):
            span = [hi[s] - lo[s] for s in range(3)]
            lo_g = jnp.where(grp == 0, lo[0], jnp.where(grp == 1, lo[1], lo[2]))
            sp_g = jnp.where(grp == 0, span[0],
                             jnp.where(grp == 1, span[1], span[2]))
            probes = lo_g + (sp_g * lane_local + 4) // 5
            probe_v[...] = jnp.clip(probes, 0, NNZ - 1)
            pltpu.async_copy(rows_hbm.at[probe_v], pval_v, sem0).wait()
            val = jnp.where(probes >= NNZ, N, pval_v[...])
            lt = (val < b_lane) & (lane_local <= 5)
            for s in range(3):
                m = jnp.sum(jnp.where((grp == s) & lt, 1, 0))
                new_lo = lo[s] + (span[s] * m + 4) // 5
                hi[s] = lo[s] + (span[s] * (m + 1) + 4) // 5
                lo[s] = new_lo
        bound = hi

        @pl.loop(0, GROUPS_PER_TILE)
        def _(cc):
            c = wid * GROUPS_PER_TILE + cc
            base_row = pl.multiple_of(c * ROWS_PER_GROUP, 8)
            start = jnp.where(cc == 0, bound[0], bound[1])
            end = jnp.where(cc == 0, bound[1], bound[2])
            pltpu.sync_copy(zeros_hbm, acc_v)

            a0 = start - (start & 7)
            nsw = (end - a0 + WBUF - 1) // WBUF

            @pl.loop(0, nsw)
            def _(w):
                k0g = pl.multiple_of(a0 + w * WBUF, 8)
                h1 = pltpu.async_copy(
                    rows_hbm.at[pl.ds(k0g, WBUF)], rows_buf, sem0)
                h2 = pltpu.async_copy(
                    cols_hbm.at[pl.ds(k0g, WBUF)], cols_buf, sems[0])
                h3 = pltpu.async_copy(
                    vals_hbm.at[pl.ds(k0g, WBUF)], vals_buf, sems[1])
                h1.wait()
                h2.wait()
                h3.wait()
                wend = jnp.minimum(end - k0g, WBUF)
                nblk = (wend + W - 1) // W

                def issue(t, b):
                    tb = jnp.minimum(t, nblk - 1)
                    src = w_hbm.at[cols_buf.at[pl.ds(tb * W, W)]]
                    pltpu.async_copy(src, g[b], sems[b])

                def wait(b):
                    pltpu.make_async_copy(
                        w_hbm.at[cols_buf.at[pl.ds(0, W)]], g[b],
                        sems[b]).wait()

                def compute(t, b):
                    tb = jnp.minimum(t, nblk - 1)
                    live = t < nblk
                    for g16 in range(W // L):
                        kbase = tb * W + g16 * L
                        rv = rows_buf[pl.ds(kbase, L)]
                        vv = vals_buf[pl.ds(kbase, L)]
                        pos = (k0g + kbase) + lane
                        valid = (pos >= start) & (pos < end) & live
                        v_eff = jnp.where(valid, vv, 0.0)
                        lr = jnp.clip(rv - base_row, 0, ROWS_PER_GROUP - 1)

                        @plsc.parallel_loop(0, L, unroll=4)
                        def _(j):
                            jf = jnp.zeros((L,), jnp.int32) + j
                            v_j = _bcast_lane(v_eff, jf)
                            lr_j = _bcast_lane(lr, jf)
                            gf = jf + g16 * L
                            for dj in range(D // L):
                                g16v = plsc.load_gather(g[b], [gf, idx1[dj]])
                                plsc.addupdate_scatter(
                                    acc_v, [lr_j, idx1[dj]], v_j * g16v)

                for b in range(NBUF):
                    issue(b, b)

                @pl.loop(0, (nblk + NBUF - 1) // NBUF)
                def _(u):
                    for b in range(NBUF):
                        t = u * NBUF + b
                        wait(b)
                        compute(t, b)
                        issue(t + NBUF, b)

                for b in range(NBUF):
                    wait(b)

            pltpu.sync_copy(acc_v, out_hbm.at[pl.ds(base_row, ROWS_PER_GROUP)])

    return sc_kernel(rows_p, cols_p, vals_p, zeros, weight)


def kernel(inp_rows, inp_cols, inp_values, weight):
    pad = NNZ_PAD - NNZ
    rows_p = jnp.pad(inp_rows, (0, pad), constant_values=N - 1)
    cols_p = jnp.pad(inp_cols, (0, pad), constant_values=0)
    vals_p = jnp.pad(inp_values, (0, pad), constant_values=0.0)
    zeros = jnp.zeros((ROWS_PER_GROUP, D), jnp.float32)
    return _sc_spmm(rows_p, cols_p, vals_p, zeros, weight)


# speedup vs baseline: 1.1912x; 1.1780x over previous
import dataclasses
import functools

import jax
import jax.numpy as jnp
from jax import lax
from jax.experimental import pallas as pl
from jax.experimental.pallas import tpu as pltpu
from jax.experimental.pallas import tpu_sc as plsc

N = 16384
NNZ = 268435
D = 256

NC = 2
NS = 16
NW = NC * NS
L = 16

ROWS_PER_GROUP = 256
NUM_GROUPS = N // ROWS_PER_GROUP
GROUPS_PER_TILE = NUM_GROUPS // NW
WBUF = 4096
W = 32
NBUF = 4
NNZ_PAD = ((NNZ + WBUF + 7) // 8) * 8

_GATHER_DNUMS = lax.GatherDimensionNumbers(
    offset_dims=(), collapsed_slice_dims=(0,), start_index_map=(0,))


def _bcast_lane(v, idx):
    return lax.gather(v, idx[:, None], _GATHER_DNUMS, (1,),
                      mode=lax.GatherScatterMode.PROMISE_IN_BOUNDS)


def _sc_spmm(rows_p, cols_p, vals_p, zeros, weight):
    mesh = plsc.VectorSubcoreMesh(core_axis_name="c", subcore_axis_name="s")
    cp = pltpu.CompilerParams()
    if "needs_layout_passes" in pltpu.CompilerParams.__dataclass_fields__:
        cp = dataclasses.replace(cp, needs_layout_passes=False)

    @functools.partial(
        pl.kernel,
        compiler_params=cp,
        out_type=jax.ShapeDtypeStruct((N, D), jnp.float32),
        mesh=mesh,
        scratch_types=[
            pltpu.VMEM((WBUF,), jnp.int32),
            pltpu.VMEM((WBUF,), jnp.int32),
            pltpu.VMEM((WBUF,), jnp.float32),
            pltpu.VMEM((L,), jnp.int32),
            pltpu.VMEM((L,), jnp.int32),
            [pltpu.VMEM((W, D), jnp.float32) for _ in range(NBUF)],
            pltpu.VMEM((ROWS_PER_GROUP, D), jnp.float32),
            pltpu.SemaphoreType.DMA,
            [pltpu.SemaphoreType.DMA for _ in range(NBUF)],
        ],
    )
    def sc_kernel(rows_hbm, cols_hbm, vals_hbm, zeros_hbm, w_hbm, out_hbm,
                  rows_buf, cols_buf, vals_buf, probe_v, pval_v, g, acc_v,
                  sem0, sems):
        wid = lax.axis_index("s") * NC + lax.axis_index("c")
        lane = lax.broadcasted_iota(jnp.int32, (L,), 0)
        zrow = jnp.zeros((L,), jnp.int32)
        idx1 = [dj * L + lane for dj in range(D // L)]

        tile_base = wid * GROUPS_PER_TILE * ROWS_PER_GROUP
        grp = jnp.minimum(lane // 5, 2)
        lane_local = lane - grp * 5 + 1
        b_lane = tile_base + grp * ROWS_PER_GROUP
        lo = [jnp.int32(-1)] * 3
        hi = [jnp.int32(NNZ)] * 3
        for _ in range(8):
            span = [hi[s] - lo[s] for s in range(3)]
            lo_g = jnp.where(grp == 0, lo[0], jnp.where(grp == 1, lo[1], lo[2]))
            sp_g = jnp.where(grp == 0, span[0],
                             jnp.where(grp == 1, span[1], span[2]))
            probes = lo_g + (sp_g * lane_local + 4) // 5
            probe_v[...] = jnp.clip(probes, 0, NNZ - 1)
            pltpu.async_copy(rows_hbm.at[probe_v], pval_v, sem0).wait()
            val = jnp.where(probes >= NNZ, N, pval_v[...])
            lt = (val < b_lane) & (lane_local <= 5)
            for s in range(3):
                m = jnp.sum(jnp.where((grp == s) & lt, 1, 0))
                new_lo = lo[s] + (span[s] * m + 4) // 5
                hi[s] = lo[s] + (span[s] * (m + 1) + 4) // 5
                lo[s] = new_lo
        bound = hi

        @pl.loop(0, GROUPS_PER_TILE)
        def _(cc):
            c = wid * GROUPS_PER_TILE + cc
            base_row = pl.multiple_of(c * ROWS_PER_GROUP, 8)
            start = jnp.where(cc == 0, bound[0], bound[1])
            end = jnp.where(cc == 0, bound[1], bound[2])
            pltpu.sync_copy(zeros_hbm, acc_v)

            a0 = start - (start & 7)
            nsw = (end - a0 + WBUF - 1) // WBUF

            @pl.loop(0, nsw)
            def _(w):
                k0g = pl.multiple_of(a0 + w * WBUF, 8)
                h1 = pltpu.async_copy(
                    rows_hbm.at[pl.ds(k0g, WBUF)], rows_buf, sem0)
                h2 = pltpu.async_copy(
                    cols_hbm.at[pl.ds(k0g, WBUF)], cols_buf, sems[0])
                h3 = pltpu.async_copy(
                    vals_hbm.at[pl.ds(k0g, WBUF)], vals_buf, sems[1])
                h1.wait()
                h2.wait()
                h3.wait()
                wend = jnp.minimum(end - k0g, WBUF)
                nblk = (wend + W - 1) // W

                def issue(t, b):
                    tb = jnp.minimum(t, nblk - 1)
                    src = w_hbm.at[cols_buf.at[pl.ds(tb * W, W)]]
                    pltpu.async_copy(src, g[b], sems[b])

                def wait(b):
                    pltpu.make_async_copy(
                        w_hbm.at[cols_buf.at[pl.ds(0, W)]],
                        g[b], sems[b]).wait()

                def compute(t, b):
                    tb = jnp.minimum(t, nblk - 1)
                    live = t < nblk
                    for g16 in range(W // L):
                        kbase = tb * W + g16 * L
                        rv = rows_buf[pl.ds(kbase, L)]
                        vv = vals_buf[pl.ds(kbase, L)]
                        pos = (k0g + kbase) + lane
                        valid = (pos >= start) & (pos < end) & live
                        v_eff = jnp.where(valid, vv, 0.0)
                        lr = jnp.clip(rv - base_row, 0, ROWS_PER_GROUP - 1)

                        @plsc.parallel_loop(0, L, unroll=2)
                        def _(j):
                            jf = jnp.zeros((L,), jnp.int32) + j
                            v_j = _bcast_lane(v_eff, jf)
                            lr_j = _bcast_lane(lr, jf)
                            gf = jf + g16 * L
                            for dj in range(D // L):
                                g16v = plsc.load_gather(g[b], [gf, idx1[dj]])
                                plsc.addupdate_scatter(
                                    acc_v, [lr_j, idx1[dj]], v_j * g16v)

                for b in range(NBUF):
                    issue(b, b)

                @pl.loop(0, (nblk + NBUF - 1) // NBUF)
                def _(u):
                    for b in range(NBUF):
                        t = u * NBUF + b
                        wait(b)
                        compute(t, b)
                        issue(t + NBUF, b)

                for b in range(NBUF):
                    wait(b)

            pltpu.sync_copy(acc_v, out_hbm.at[pl.ds(base_row, ROWS_PER_GROUP)])

    return sc_kernel(rows_p, cols_p, vals_p, zeros, weight)


def kernel(inp_rows, inp_cols, inp_values, weight):
    pad = NNZ_PAD - NNZ
    rows_p = jnp.pad(inp_rows, (0, pad), constant_values=N - 1)
    cols_p = jnp.pad(inp_cols, (0, pad), constant_values=0)
    vals_p = jnp.pad(inp_values, (0, pad), constant_values=0.0)
    zeros = jnp.zeros((ROWS_PER_GROUP, D), jnp.float32)
    return _sc_spmm(rows_p, cols_p, vals_p, zeros, weight)


# speedup vs baseline: 1.4374x; 1.2067x over previous
import dataclasses
import functools

import jax
import jax.numpy as jnp
from jax import lax
from jax.experimental import pallas as pl
from jax.experimental.pallas import tpu as pltpu
from jax.experimental.pallas import tpu_sc as plsc

N = 16384
NNZ = 268435
D = 256

NC = 2
NS = 16
NW = NC * NS
L = 16

ROWS_PER_GROUP = 256
NUM_GROUPS = N // ROWS_PER_GROUP
GROUPS_PER_TILE = NUM_GROUPS // NW
WBUF = 4096
W = 32
NBUF = 4
NNZ_PAD = ((NNZ + WBUF + 7) // 8) * 8

_GATHER_DNUMS = lax.GatherDimensionNumbers(
    offset_dims=(), collapsed_slice_dims=(0,), start_index_map=(0,))


def _bcast_lane(v, idx):
    return lax.gather(v, idx[:, None], _GATHER_DNUMS, (1,),
                      mode=lax.GatherScatterMode.PROMISE_IN_BOUNDS)


def _sc_spmm(rows_p, cols_p, vals_p, zeros, weight):
    mesh = plsc.VectorSubcoreMesh(core_axis_name="c", subcore_axis_name="s")
    cp = pltpu.CompilerParams()
    if "needs_layout_passes" in pltpu.CompilerParams.__dataclass_fields__:
        cp = dataclasses.replace(cp, needs_layout_passes=False)

    @functools.partial(
        pl.kernel,
        compiler_params=cp,
        out_type=jax.ShapeDtypeStruct((N, D), jnp.float32),
        mesh=mesh,
        scratch_types=[
            pltpu.VMEM((WBUF,), jnp.int32),
            pltpu.VMEM((WBUF,), jnp.int32),
            pltpu.VMEM((WBUF,), jnp.float32),
            pltpu.VMEM((L,), jnp.int32),
            pltpu.VMEM((L,), jnp.int32),
            [pltpu.VMEM((W, D), jnp.float32) for _ in range(NBUF)],
            pltpu.VMEM((ROWS_PER_GROUP, D), jnp.float32),
            pltpu.SemaphoreType.DMA,
            [pltpu.SemaphoreType.DMA for _ in range(NBUF)],
        ],
    )
    def sc_kernel(rows_hbm, cols_hbm, vals_hbm, zeros_hbm, w_hbm, out_hbm,
                  rows_buf, cols_buf, vals_buf, probe_v, pval_v, g, acc_v,
                  sem0, sems):
        wid = lax.axis_index("s") * NC + lax.axis_index("c")
        lane = lax.broadcasted_iota(jnp.int32, (L,), 0)
        zrow = jnp.zeros((L,), jnp.int32)
        idx1 = [dj * L + lane for dj in range(D // L)]

        tile_base = wid * GROUPS_PER_TILE * ROWS_PER_GROUP
        grp = jnp.minimum(lane // 5, 2)
        lane_local = lane - grp * 5 + 1
        b_lane = tile_base + grp * ROWS_PER_GROUP
        lo = [jnp.int32(-1)] * 3
        hi = [jnp.int32(NNZ)] * 3
        for _ in range(8):
            span = [hi[s] - lo[s] for s in range(3)]
            lo_g = jnp.where(grp == 0, lo[0], jnp.where(grp == 1, lo[1], lo[2]))
            sp_g = jnp.where(grp == 0, span[0],
                             jnp.where(grp == 1, span[1], span[2]))
            probes = lo_g + (sp_g * lane_local + 4) // 5
            probe_v[...] = jnp.clip(probes, 0, NNZ - 1)
            pltpu.async_copy(rows_hbm.at[probe_v], pval_v, sem0).wait()
            val = jnp.where(probes >= NNZ, N, pval_v[...])
            lt = (val < b_lane) & (lane_local <= 5)
            for s in range(3):
                m = jnp.sum(jnp.where((grp == s) & lt, 1, 0))
                new_lo = lo[s] + (span[s] * m + 4) // 5
                hi[s] = lo[s] + (span[s] * (m + 1) + 4) // 5
                lo[s] = new_lo
        bound = hi

        @pl.loop(0, GROUPS_PER_TILE)
        def _(cc):
            c = wid * GROUPS_PER_TILE + cc
            base_row = pl.multiple_of(c * ROWS_PER_GROUP, 8)
            start = jnp.where(cc == 0, bound[0], bound[1])
            end = jnp.where(cc == 0, bound[1], bound[2])
            pltpu.sync_copy(zeros_hbm, acc_v)

            a0 = start - (start & 7)
            nsw = (end - a0 + WBUF - 1) // WBUF

            @pl.loop(0, nsw)
            def _(w):
                k0g = pl.multiple_of(a0 + w * WBUF, 8)
                h1 = pltpu.async_copy(
                    rows_hbm.at[pl.ds(k0g, WBUF)], rows_buf, sem0)
                h2 = pltpu.async_copy(
                    cols_hbm.at[pl.ds(k0g, WBUF)], cols_buf, sems[0])
                h3 = pltpu.async_copy(
                    vals_hbm.at[pl.ds(k0g, WBUF)], vals_buf, sems[1])
                h1.wait()
                h2.wait()
                h3.wait()
                wend = jnp.minimum(end - k0g, WBUF)
                nblk = (wend + W - 1) // W

                def issue(t, b):
                    tb = jnp.minimum(t, nblk - 1)
                    src = w_hbm.at[cols_buf.at[pl.ds(tb * W, W)]]
                    pltpu.async_copy(src, g[b], sems[b])

                def wait(b):
                    pltpu.make_async_copy(
                        w_hbm.at[cols_buf.at[pl.ds(0, W)]],
                        g[b], sems[b]).wait()

                def compute(t, b):
                    tb = jnp.minimum(t, nblk - 1)
                    live = t < nblk
                    for g16 in range(W // L):
                        kbase = tb * W + g16 * L
                        rv = rows_buf[pl.ds(kbase, L)]
                        vv = vals_buf[pl.ds(kbase, L)]
                        pos = (k0g + kbase) + lane
                        valid = (pos >= start) & (pos < end) & live
                        v_eff = jnp.where(valid, vv, 0.0)
                        lr = jnp.clip(rv - base_row, 0, ROWS_PER_GROUP - 1)

                        @plsc.parallel_loop(0, L, unroll=1)
                        def _(j):
                            jf = jnp.zeros((L,), jnp.int32) + j
                            v_j = _bcast_lane(v_eff, jf)
                            lr_j = _bcast_lane(lr, jf)
                            gf = jf + g16 * L
                            for dj in range(D // L):
                                g16v = plsc.load_gather(g[b], [gf, idx1[dj]])
                                plsc.addupdate_scatter(
                                    acc_v, [lr_j, idx1[dj]], v_j * g16v)

                for b in range(NBUF):
                    issue(b, b)

                @pl.loop(0, (nblk + NBUF - 1) // NBUF)
                def _(u):
                    for b in range(NBUF):
                        t = u * NBUF + b
                        wait(b)
                        compute(t, b)
                        issue(t + NBUF, b)

                for b in range(NBUF):
                    wait(b)

            pltpu.sync_copy(acc_v, out_hbm.at[pl.ds(base_row, ROWS_PER_GROUP)])

    return sc_kernel(rows_p, cols_p, vals_p, zeros, weight)


def kernel(inp_rows, inp_cols, inp_values, weight):
    pad = NNZ_PAD - NNZ
    rows_p = jnp.pad(inp_rows, (0, pad), constant_values=N - 1)
    cols_p = jnp.pad(inp_cols, (0, pad), constant_values=0)
    vals_p = jnp.pad(inp_values, (0, pad), constant_values=0.0)
    zeros = jnp.zeros((ROWS_PER_GROUP, D), jnp.float32)
    return _sc_spmm(rows_p, cols_p, vals_p, zeros, weight)
